# baseline (speedup 1.0000x reference)
import functools

import jax
import jax.numpy as jnp
from jax import lax
from jax.experimental import pallas as pl
from jax.experimental.pallas import tpu as pltpu
from jax.experimental.pallas import tpu_sc as plsc

NC = 2
NS = 16
NW = NC * NS
L = 16
PACK = 4


def _rsqrt_nr(x):
    xi = plsc.bitcast(x, jnp.int32)
    yi = jnp.int32(0x5F3759DF) - (xi >> 1)
    y = plsc.bitcast(yi, jnp.float32)
    for _ in range(3):
        y = y * (jnp.float32(1.5) - jnp.float32(0.5) * x * y * y)
    return y


def _make_sc_call(B, D):
    b_per_w = B // NW
    half = b_per_w // 2
    mesh = plsc.VectorSubcoreMesh(
        core_axis_name="c", subcore_axis_name="s", num_cores=NC, num_subcores=NS
    )
    w = PACK * D

    @functools.partial(
        pl.kernel,
        out_type=jax.ShapeDtypeStruct((B,), jnp.float32),
        mesh=mesh,
        compiler_params=pltpu.CompilerParams(
            needs_layout_passes=False, use_tc_tiling_on_sc=False),
        scratch_types=[
            pltpu.VMEM((b_per_w,), jnp.int32),
            pltpu.VMEM((b_per_w,), jnp.int32),
            pltpu.VMEM((b_per_w,), jnp.int32),
            pltpu.VMEM((b_per_w,), jnp.int32),
            pltpu.VMEM((half, w), jnp.float32),
            pltpu.VMEM((half, w), jnp.float32),
            pltpu.VMEM((b_per_w,), jnp.float32),
            pltpu.SemaphoreType.DMA,
            pltpu.SemaphoreType.DMA,
        ],
    )
    def sc_call(uid_hbm, iid_hbm, ut_hbm, it_hbm, out_hbm,
                uidx_v, iidx_v, usr_v, isr_v, ubuf_v, ibuf_v, res_v,
                usem, isem):
        wid = lax.axis_index("s") * NC + lax.axis_index("c")
        base = wid * b_per_w

        pltpu.sync_copy(uid_hbm.at[pl.ds(base, b_per_w)], uidx_v)
        pltpu.sync_copy(iid_hbm.at[pl.ds(base, b_per_w)], iidx_v)

        for c in range(b_per_w // L):
            sl = pl.ds(c * L, L)
            usr_v[sl] = uidx_v[sl] >> 2
            isr_v[sl] = iidx_v[sl] >> 2

        for h in range(2):
            off = h * half
            cu = pltpu.async_copy(
                ut_hbm.at[usr_v.at[pl.ds(off, half)]], ubuf_v, usem)
            ci = pltpu.async_copy(
                it_hbm.at[isr_v.at[pl.ds(off, half)]], ibuf_v, isem)
            cu.wait()
            ci.wait()

            def group_body(g, _):
                rows = g * L + lax.iota(jnp.int32, L)
                gsl = pl.ds(off + g * L, L)
                ucol = (uidx_v[gsl] & 3) * D
                icol = (iidx_v[gsl] & 3) * D
                dot = jnp.zeros((L,), jnp.float32)
                nu2 = jnp.zeros((L,), jnp.float32)
                ni2 = jnp.zeros((L,), jnp.float32)
                for d in range(D):
                    u = plsc.load_gather(ubuf_v, [rows, ucol + d])
                    v = plsc.load_gather(ibuf_v, [rows, icol + d])
                    dot = dot + u * v
                    nu2 = nu2 + u * u
                    ni2 = ni2 + v * v
                rnu = _rsqrt_nr(jnp.maximum(nu2, jnp.float32(1e-16)))
                rni = _rsqrt_nr(jnp.maximum(ni2, jnp.float32(1e-16)))
                res_v[gsl] = dot * rnu * rni
                return 0

            lax.fori_loop(0, half // L, group_body, 0)

        pltpu.sync_copy(res_v, out_hbm.at[pl.ds(base, b_per_w)])

    return sc_call


def kernel(user_id, item_id, user_table, item_table):
    B = user_id.shape[0]
    N, D = user_table.shape
    uid = user_id.astype(jnp.int32)
    iid = item_id.astype(jnp.int32)
    ut = user_table.reshape(N // PACK, PACK * D)
    it = item_table.reshape(N // PACK, PACK * D)
    return _make_sc_call(B, D)(uid, iid, ut, it)
